# baseline (speedup 1.0000x reference)
import jax
import jax.numpy as jnp
from jax import lax
from jax.experimental import pallas as pl
from jax.experimental.pallas import tpu as pltpu
from jax.experimental.pallas import tpu_sc as plsc

CAPACITY, GROUPS, HEAD_DIM = 8192, 8, 128
NEW_TOKENS = 32

NC, NS = 2, 16
NW = NC * NS
RPW = CAPACITY // NW
CH = 16
NCHUNK = RPW // CH
NB = 4


def _sc_body(k_ref, v_ref, ak_ref, av_ref, cur_ref, ok_ref, ov_ref,
             buf, curbuf, sem_in, sem_out):
    wid = lax.axis_index("s") * NC + lax.axis_index("c")
    base = wid * RPW

    pltpu.sync_copy(cur_ref, curbuf)
    cur = curbuf[...][0]
    cur = jnp.clip(cur, 0, CAPACITY - NEW_TOKENS)
    cur = (cur // 8) * 8

    ops = ([(k_ref, ok_ref, i) for i in range(NCHUNK)]
           + [(v_ref, ov_ref, i) for i in range(NCHUNK)])
    nop = len(ops)

    def in_copy(j):
        src, _, i = ops[j]
        rows = pl.ds(base + i * CH, CH)
        return pltpu.make_async_copy(src.at[rows], buf.at[j % NB],
                                     sem_in.at[j % NB])

    def out_copy(j):
        _, dst, i = ops[j]
        rows = pl.ds(base + i * CH, CH)
        return pltpu.make_async_copy(buf.at[j % NB], dst.at[rows],
                                     sem_out.at[j % NB])

    in_copy(0).start()
    for j in range(nop):
        in_copy(j).wait()
        out_copy(j).start()
        if j + 1 < nop:
            if j + 1 >= NB:
                out_copy(j + 1 - NB).wait()
            in_copy(j + 1).start()
    for j in range(nop - NB, nop):
        out_copy(j).wait()

    w0 = cur // RPW
    off = cur - w0 * RPW
    contained = off <= RPW - NEW_TOKENS

    PC = min(CH, NEW_TOKENS)

    @pl.when(jnp.logical_and(contained, wid == w0))
    def _():
        for h in range(0, NEW_TOKENS, PC):
            b = (h // PC) % NB
            pltpu.sync_copy(ak_ref.at[pl.ds(h, PC)], buf.at[b, pl.ds(0, PC)])
            pltpu.sync_copy(buf.at[b, pl.ds(0, PC)],
                            ok_ref.at[pl.ds(cur + h, PC)])
            pltpu.sync_copy(av_ref.at[pl.ds(h, PC)], buf.at[b, pl.ds(0, PC)])
            pltpu.sync_copy(buf.at[b, pl.ds(0, PC)],
                            ov_ref.at[pl.ds(cur + h, PC)])

    @pl.when(jnp.logical_not(contained))
    def _():
        for a in range(0, NEW_TOKENS, 8):
            r = ((cur + a) // 8) * 8

            @pl.when(r // RPW == wid)
            def _():
                rb = buf.at[0, pl.ds(0, 8)]
                pltpu.sync_copy(ak_ref.at[pl.ds(a, 8)], rb)
                pltpu.sync_copy(rb, ok_ref.at[pl.ds(r, 8)])
                pltpu.sync_copy(av_ref.at[pl.ds(a, 8)], rb)
                pltpu.sync_copy(rb, ov_ref.at[pl.ds(r, 8)])


def kernel(keys, values, added_keys, added_values, current_length):
    num_added = added_keys.shape[0]
    cur16 = jnp.full((16,), current_length, dtype=jnp.int32)

    sc = pl.kernel(
        _sc_body,
        out_type=(
            jax.ShapeDtypeStruct((CAPACITY, GROUPS, HEAD_DIM), jnp.float32),
            jax.ShapeDtypeStruct((CAPACITY, GROUPS, HEAD_DIM), jnp.float32),
        ),
        mesh=plsc.VectorSubcoreMesh(core_axis_name="c", subcore_axis_name="s"),
        scratch_types=[
            pltpu.VMEM((NB, CH, GROUPS, HEAD_DIM), jnp.float32),
            pltpu.VMEM((16,), jnp.int32),
            pltpu.SemaphoreType.DMA((NB,)),
            pltpu.SemaphoreType.DMA((NB,)),
        ],
    )
    ok, ov = sc(keys, values, added_keys, added_values, cur16)
    return ok, ov, current_length + num_added


# speedup vs baseline: 1.1947x; 1.1947x over previous
import jax
import jax.numpy as jnp
from jax import lax
from jax.experimental import pallas as pl
from jax.experimental.pallas import tpu as pltpu
from jax.experimental.pallas import tpu_sc as plsc

CAPACITY, GROUPS, HEAD_DIM = 8192, 8, 128
NEW_TOKENS = 32

NC, NS = 2, 16
NW = NC * NS
RPW = CAPACITY // NW
CH = 32
NCHUNK = RPW // CH
NB = 3


def _sc_body(k_ref, v_ref, ak_ref, av_ref, cur_ref, ok_ref, ov_ref,
             buf, curbuf, sem_in, sem_out):
    wid = lax.axis_index("s") * NC + lax.axis_index("c")
    base = wid * RPW

    pltpu.sync_copy(cur_ref, curbuf)
    cur = curbuf[...][0]
    cur = jnp.clip(cur, 0, CAPACITY - NEW_TOKENS)
    cur = (cur // 8) * 8

    ops = ([(k_ref, ok_ref, i) for i in range(NCHUNK)]
           + [(v_ref, ov_ref, i) for i in range(NCHUNK)])
    nop = len(ops)

    def in_copy(j):
        src, _, i = ops[j]
        rows = pl.ds(base + i * CH, CH)
        return pltpu.make_async_copy(src.at[rows], buf.at[j % NB],
                                     sem_in.at[j % NB])

    def out_copy(j):
        _, dst, i = ops[j]
        rows = pl.ds(base + i * CH, CH)
        return pltpu.make_async_copy(buf.at[j % NB], dst.at[rows],
                                     sem_out.at[j % NB])

    in_copy(0).start()
    for j in range(nop):
        in_copy(j).wait()
        out_copy(j).start()
        if j + 1 < nop:
            if j + 1 >= NB:
                out_copy(j + 1 - NB).wait()
            in_copy(j + 1).start()
    for j in range(nop - NB, nop):
        out_copy(j).wait()

    w0 = cur // RPW
    off = cur - w0 * RPW
    contained = off <= RPW - NEW_TOKENS

    PC = min(CH, NEW_TOKENS)

    @pl.when(jnp.logical_and(contained, wid == w0))
    def _():
        for h in range(0, NEW_TOKENS, PC):
            b = (h // PC) % NB
            pltpu.sync_copy(ak_ref.at[pl.ds(h, PC)], buf.at[b, pl.ds(0, PC)])
            pltpu.sync_copy(buf.at[b, pl.ds(0, PC)],
                            ok_ref.at[pl.ds(cur + h, PC)])
            pltpu.sync_copy(av_ref.at[pl.ds(h, PC)], buf.at[b, pl.ds(0, PC)])
            pltpu.sync_copy(buf.at[b, pl.ds(0, PC)],
                            ov_ref.at[pl.ds(cur + h, PC)])

    @pl.when(jnp.logical_not(contained))
    def _():
        for a in range(0, NEW_TOKENS, 8):
            r = ((cur + a) // 8) * 8

            @pl.when(r // RPW == wid)
            def _():
                rb = buf.at[0, pl.ds(0, 8)]
                pltpu.sync_copy(ak_ref.at[pl.ds(a, 8)], rb)
                pltpu.sync_copy(rb, ok_ref.at[pl.ds(r, 8)])
                pltpu.sync_copy(av_ref.at[pl.ds(a, 8)], rb)
                pltpu.sync_copy(rb, ov_ref.at[pl.ds(r, 8)])


def kernel(keys, values, added_keys, added_values, current_length):
    num_added = added_keys.shape[0]
    cur16 = jnp.full((16,), current_length, dtype=jnp.int32)

    sc = pl.kernel(
        _sc_body,
        out_type=(
            jax.ShapeDtypeStruct((CAPACITY, GROUPS, HEAD_DIM), jnp.float32),
            jax.ShapeDtypeStruct((CAPACITY, GROUPS, HEAD_DIM), jnp.float32),
        ),
        mesh=plsc.VectorSubcoreMesh(core_axis_name="c", subcore_axis_name="s"),
        scratch_types=[
            pltpu.VMEM((NB, CH, GROUPS, HEAD_DIM), jnp.float32),
            pltpu.VMEM((16,), jnp.int32),
            pltpu.SemaphoreType.DMA((NB,)),
            pltpu.SemaphoreType.DMA((NB,)),
        ],
    )
    ok, ov = sc(keys, values, added_keys, added_values, cur16)
    return ok, ov, current_length + num_added


# speedup vs baseline: 1.2113x; 1.0139x over previous
import jax
import jax.numpy as jnp
from jax import lax
from jax.experimental import pallas as pl
from jax.experimental.pallas import tpu as pltpu
from jax.experimental.pallas import tpu_sc as plsc

CAPACITY, GROUPS, HEAD_DIM = 8192, 8, 128
NEW_TOKENS = 32

NC, NS = 2, 16
NW = NC * NS
RPW = CAPACITY // NW
CH = 32
NCHUNK = RPW // CH
NB = 3


def _sc_body(k_ref, v_ref, ak_ref, av_ref, cur_ref, ok_ref, ov_ref,
             buf, curbuf, sem_in, sem_out):
    wid = lax.axis_index("s") * NC + lax.axis_index("c")
    base = wid * RPW

    pltpu.sync_copy(cur_ref, curbuf)
    cur = curbuf[...][0]
    cur = jnp.clip(cur, 0, CAPACITY - NEW_TOKENS)
    cur = (cur // 8) * 8

    ops = ([(k_ref, ok_ref, i) for i in range(NCHUNK)]
           + [(v_ref, ov_ref, i) for i in range(NCHUNK)])
    nop = len(ops)

    def in_copy(j):
        src, _, i = ops[j]
        rows = pl.ds(base + i * CH, CH)
        return pltpu.make_async_copy(src.at[rows], buf.at[j % NB],
                                     sem_in.at[j % NB])

    def out_copy(j):
        _, dst, i = ops[j]
        rows = pl.ds(base + i * CH, CH)
        return pltpu.make_async_copy(buf.at[j % NB], dst.at[rows],
                                     sem_out.at[j % NB])

    for b in range(min(NB, nop)):
        in_copy(b).start()
    for j in range(nop):
        in_copy(j).wait()
        out_copy(j).start()
        if j + NB < nop:
            out_copy(j).wait()
            in_copy(j + NB).start()
    for j in range(max(0, nop - NB), nop):
        out_copy(j).wait()

    w0 = cur // RPW
    off = cur - w0 * RPW
    contained = off <= RPW - NEW_TOKENS

    PC = min(CH, NEW_TOKENS)

    @pl.when(jnp.logical_and(contained, wid == w0))
    def _():
        for h in range(0, NEW_TOKENS, PC):
            b = (h // PC) % NB
            pltpu.sync_copy(ak_ref.at[pl.ds(h, PC)], buf.at[b, pl.ds(0, PC)])
            pltpu.sync_copy(buf.at[b, pl.ds(0, PC)],
                            ok_ref.at[pl.ds(cur + h, PC)])
            pltpu.sync_copy(av_ref.at[pl.ds(h, PC)], buf.at[b, pl.ds(0, PC)])
            pltpu.sync_copy(buf.at[b, pl.ds(0, PC)],
                            ov_ref.at[pl.ds(cur + h, PC)])

    @pl.when(jnp.logical_not(contained))
    def _():
        for a in range(0, NEW_TOKENS, 8):
            r = ((cur + a) // 8) * 8

            @pl.when(r // RPW == wid)
            def _():
                rb = buf.at[0, pl.ds(0, 8)]
                pltpu.sync_copy(ak_ref.at[pl.ds(a, 8)], rb)
                pltpu.sync_copy(rb, ok_ref.at[pl.ds(r, 8)])
                pltpu.sync_copy(av_ref.at[pl.ds(a, 8)], rb)
                pltpu.sync_copy(rb, ov_ref.at[pl.ds(r, 8)])


def kernel(keys, values, added_keys, added_values, current_length):
    num_added = added_keys.shape[0]
    cur16 = jnp.full((16,), current_length, dtype=jnp.int32)

    sc = pl.kernel(
        _sc_body,
        out_type=(
            jax.ShapeDtypeStruct((CAPACITY, GROUPS, HEAD_DIM), jnp.float32),
            jax.ShapeDtypeStruct((CAPACITY, GROUPS, HEAD_DIM), jnp.float32),
        ),
        mesh=plsc.VectorSubcoreMesh(core_axis_name="c", subcore_axis_name="s"),
        scratch_types=[
            pltpu.VMEM((NB, CH, GROUPS, HEAD_DIM), jnp.float32),
            pltpu.VMEM((16,), jnp.int32),
            pltpu.SemaphoreType.DMA((NB,)),
            pltpu.SemaphoreType.DMA((NB,)),
        ],
    )
    ok, ov = sc(keys, values, added_keys, added_values, cur16)
    return ok, ov, current_length + num_added


# speedup vs baseline: 1.2874x; 1.0628x over previous
import jax
import jax.numpy as jnp
from jax import lax
from jax.experimental import pallas as pl
from jax.experimental.pallas import tpu as pltpu
from jax.experimental.pallas import tpu_sc as plsc

CAPACITY, GROUPS, HEAD_DIM = 8192, 8, 128
NEW_TOKENS = 32

NC, NS = 2, 16
NW = NC * NS
RPW = CAPACITY // NW
CH = 32
NCHUNK = RPW // CH
NB = 3


def _sc_bulk_copy(k_ref, v_ref, ok_ref, ov_ref, buf, sem_in, sem_out):
    wid = lax.axis_index("s") * NC + lax.axis_index("c")
    base = wid * RPW

    ops = ([(k_ref, ok_ref, i) for i in range(NCHUNK)]
           + [(v_ref, ov_ref, i) for i in range(NCHUNK)])
    nop = len(ops)

    def in_copy(j):
        src, _, i = ops[j]
        rows = pl.ds(base + i * CH, CH)
        return pltpu.make_async_copy(src.at[rows], buf.at[j % NB],
                                     sem_in.at[j % NB])

    def out_copy(j):
        _, dst, i = ops[j]
        rows = pl.ds(base + i * CH, CH)
        return pltpu.make_async_copy(buf.at[j % NB], dst.at[rows],
                                     sem_out.at[j % NB])

    for b in range(min(NB, nop)):
        in_copy(b).start()
    for j in range(nop):
        in_copy(j).wait()
        out_copy(j).start()
        if j + NB < nop:
            out_copy(j).wait()
            in_copy(j + NB).start()
    for j in range(max(0, nop - NB), nop):
        out_copy(j).wait()


def _tc_slab_update(cur_ref, ok_in, ov_in, ak_ref, av_ref, ok_ref, ov_ref,
                    akbuf, avbuf, sems):
    del ok_in, ov_in
    cur = jnp.clip(cur_ref[0], 0, CAPACITY - NEW_TOKENS)
    cur = pl.multiple_of(cur, 8)

    ins = [pltpu.make_async_copy(ak_ref, akbuf, sems.at[0]),
           pltpu.make_async_copy(av_ref, avbuf, sems.at[1])]
    for c in ins:
        c.start()
    for c in ins:
        c.wait()
    sl = pl.ds(cur, NEW_TOKENS)
    outs = [pltpu.make_async_copy(akbuf, ok_ref.at[sl], sems.at[2]),
            pltpu.make_async_copy(avbuf, ov_ref.at[sl], sems.at[3])]
    for c in outs:
        c.start()
    for c in outs:
        c.wait()


def kernel(keys, values, added_keys, added_values, current_length):
    num_added = added_keys.shape[0]
    cur1 = jnp.reshape(current_length, (1,)).astype(jnp.int32)
    shp = (
        jax.ShapeDtypeStruct((CAPACITY, GROUPS, HEAD_DIM), jnp.float32),
        jax.ShapeDtypeStruct((CAPACITY, GROUPS, HEAD_DIM), jnp.float32),
    )

    sc = pl.kernel(
        _sc_bulk_copy,
        out_type=shp,
        mesh=plsc.VectorSubcoreMesh(core_axis_name="c", subcore_axis_name="s"),
        scratch_types=[
            pltpu.VMEM((NB, CH, GROUPS, HEAD_DIM), jnp.float32),
            pltpu.SemaphoreType.DMA((NB,)),
            pltpu.SemaphoreType.DMA((NB,)),
        ],
    )
    bk, bv = sc(keys, values)

    hbm = pl.BlockSpec(memory_space=pltpu.MemorySpace.HBM)
    ok, ov = pl.pallas_call(
        _tc_slab_update,
        in_specs=[pl.BlockSpec(memory_space=pltpu.SMEM), hbm, hbm, hbm, hbm],
        out_specs=(hbm, hbm),
        out_shape=shp,
        input_output_aliases={1: 0, 2: 1},
        scratch_shapes=[
            pltpu.VMEM((NEW_TOKENS, GROUPS, HEAD_DIM), jnp.float32),
            pltpu.VMEM((NEW_TOKENS, GROUPS, HEAD_DIM), jnp.float32),
            pltpu.SemaphoreType.DMA((4,)),
        ],
    )(cur1, bk, bv, added_keys, added_values)
    return ok, ov, current_length + num_added
